# baseline (speedup 1.0000x reference)
import functools

import jax
import jax.numpy as jnp
from jax import lax
from jax.experimental import pallas as pl
from jax.experimental.pallas import tpu as pltpu
from jax.experimental.pallas import tpu_sc as plsc

N_WORKERS = 32
CHUNK = 4096


def _build_colgather(B, F, V, D):
    n_pairs = F * D
    pairs_per_w = n_pairs // N_WORKERS
    n_chunks = B // CHUNK
    mesh = plsc.VectorSubcoreMesh(core_axis_name="c", subcore_axis_name="s")

    @functools.partial(
        pl.kernel,
        mesh=mesh,
        out_type=jax.ShapeDtypeStruct((n_pairs, B), jnp.float32),
        scratch_types=[
            pltpu.VMEM(((V + 127) // 128 * 128,), jnp.float32),
            pltpu.VMEM((B,), jnp.int32),
            pltpu.VMEM((2, CHUNK), jnp.float32),
            pltpu.SemaphoreType.DMA,
            pltpu.SemaphoreType.DMA((2,)),
        ],
        compiler_params=pltpu.CompilerParams(needs_layout_passes=False),
    )
    def colgather_kernel(xt_hbm, tabt_hbm, tail_hbm, out_hbm, col_v, idx_v, res_v, semc, sems):
        sid = lax.axis_index("s")
        wid = sid * 2 + lax.axis_index("c")

        @pl.when(sid % 2 == 1)
        def _():
            pl.delay(3000)

        def store_cp(g, c, b):
            return pltpu.make_async_copy(
                res_v.at[b], out_hbm.at[g, pl.ds(c * CHUNK, CHUNK)], sems.at[b]
            )

        def pair_body(j, f_prev):
            g = wid * pairs_per_w + j
            f = g // D
            row_ref = tabt_hbm.at[g]
            main = V // 128 * 128
            q4 = (main // 4) // 128 * 128
            bounds = [0, q4, 2 * q4, 3 * q4, main]
            col_cps = [
                pltpu.make_async_copy(
                    row_ref.at[pl.ds(bounds[q], bounds[q + 1] - bounds[q])],
                    col_v.at[pl.ds(bounds[q], bounds[q + 1] - bounds[q])],
                    semc,
                )
                for q in range(4)
            ]
            for cp in col_cps:
                cp.start()
            if main < V:
                pltpu.sync_copy(tail_hbm.at[g], col_v.at[pl.ds(main, 128)])

            @pl.when(f != f_prev)
            def _():
                pltpu.sync_copy(xt_hbm.at[f], idx_v)

            for cp in col_cps:
                cp.wait()
            for c in range(n_chunks):
                b = c % 2

                @pl.when(j * n_chunks + c >= 2)
                def _():
                    store_cp(g, c, b).wait()

                @plsc.parallel_loop(0, CHUNK, step=16, unroll=16)
                def _(i):
                    idx16 = idx_v[pl.ds(c * CHUNK + i, 16)]
                    res_v[b, pl.ds(i, 16)] = plsc.load_gather(col_v, [idx16])

                store_cp(g, c, b).start()
            return f

        f_last = lax.fori_loop(0, pairs_per_w, pair_body, jnp.int32(-1))
        g_last = wid * pairs_per_w + pairs_per_w - 1
        for c in (n_chunks - 2, n_chunks - 1):
            store_cp(g_last, c, c % 2).wait()

    return colgather_kernel


def kernel(x, tables):
    B, F = x.shape
    _, V, D = tables.shape
    tabt = tables.transpose(0, 2, 1).reshape(F * D, V)
    xt = x.T
    main = V // 128 * 128
    tail = jnp.pad(tabt[:, main:], ((0, 0), (0, 128 - (V - main))))
    out = _build_colgather(B, F, V, D)(xt, tabt, tail)
    return out.T


# speedup vs baseline: 1.0010x; 1.0010x over previous
import functools

import jax
import jax.numpy as jnp
from jax import lax
from jax.experimental import pallas as pl
from jax.experimental.pallas import tpu as pltpu
from jax.experimental.pallas import tpu_sc as plsc

N_WORKERS = 32
CHUNK = 4096


def _build_colgather(B, F, V, D):
    n_pairs = F * D
    pairs_per_w = n_pairs // N_WORKERS
    n_chunks = B // CHUNK
    mesh = plsc.VectorSubcoreMesh(core_axis_name="c", subcore_axis_name="s")

    @functools.partial(
        pl.kernel,
        mesh=mesh,
        out_type=jax.ShapeDtypeStruct((n_pairs, B), jnp.float32),
        scratch_types=[
            pltpu.VMEM((V,), jnp.float32),
            pltpu.VMEM((B,), jnp.int32),
            pltpu.VMEM((2, CHUNK), jnp.float32),
            pltpu.SemaphoreType.DMA,
            pltpu.SemaphoreType.DMA((2,)),
        ],
        compiler_params=pltpu.CompilerParams(needs_layout_passes=False),
    )
    def colgather_kernel(xt_hbm, tabt_hbm, out_hbm, col_v, idx_v, res_v, semc, sems):
        wid = lax.axis_index("s") * 2 + lax.axis_index("c")

        def store_cp(g, c, b):
            return pltpu.make_async_copy(
                res_v.at[b], out_hbm.at[g, pl.ds(c * CHUNK, CHUNK)], sems.at[b]
            )

        def pair_body(j, f_prev):
            g = wid * pairs_per_w + j
            f = g // D
            col_cp = pltpu.make_async_copy(tabt_hbm.at[g], col_v, semc)
            col_cp.start()

            @pl.when(f != f_prev)
            def _():
                pltpu.sync_copy(xt_hbm.at[f], idx_v)

            col_cp.wait()
            for c in range(n_chunks):
                b = c % 2

                @pl.when(j * n_chunks + c >= 2)
                def _():
                    store_cp(g, c, b).wait()

                @plsc.parallel_loop(0, CHUNK, step=16, unroll=8)
                def _(i):
                    idx16 = idx_v[pl.ds(c * CHUNK + i, 16)]
                    res_v[b, pl.ds(i, 16)] = plsc.load_gather(col_v, [idx16])

                store_cp(g, c, b).start()
            return f

        lax.fori_loop(0, pairs_per_w, pair_body, jnp.int32(-1))
        g_last = wid * pairs_per_w + pairs_per_w - 1
        for c in (n_chunks - 2, n_chunks - 1):
            store_cp(g_last, c, c % 2).wait()

    return colgather_kernel


def kernel(x, tables):
    B, F = x.shape
    _, V, D = tables.shape
    tabt = tables.transpose(0, 2, 1).reshape(F * D, V)
    xt = x.T
    out = _build_colgather(B, F, V, D)(xt, tabt)
    return out.T
